# baseline (speedup 1.0000x reference)
import functools

import jax
import jax.numpy as jnp
from jax import lax
from jax.experimental import pallas as pl
from jax.experimental.pallas import tpu as pltpu
import jax.experimental.pallas.tpu_sc as plsc

NC = 2
NS = 16
L = 16
NW = NC * NS

RD = 32
NT = 7


@functools.partial(jax.jit, static_argnums=(0,))
def _combine(B, tabs, idxs, w_bcast):
    b_per_w = B // NW
    mesh = plsc.VectorSubcoreMesh(core_axis_name="c", subcore_axis_name="s")

    @functools.partial(
        pl.kernel,
        out_type=jax.ShapeDtypeStruct((B, RD), jnp.float32),
        mesh=mesh,
        scratch_types=(
            [pltpu.VMEM((b_per_w,), jnp.int32) for _ in range(NT)]
            + [pltpu.VMEM((b_per_w, RD), jnp.float32) for _ in range(NT)]
            + [pltpu.VMEM((8, L), jnp.float32), pltpu.SemaphoreType.DMA]
        ),
        compiler_params=pltpu.CompilerParams(use_tc_tiling_on_sc=False),
    )
    def k(t0, t1, t2, t3, t4, t5, t6, i0, i1, i2, i3, i4, i5, i6, w_hbm,
          out_hbm, x0, x1, x2, x3, x4, x5, x6, r0, r1, r2, r3, r4, r5, r6,
          w_v, sem):
        tab_refs = (t0, t1, t2, t3, t4, t5, t6)
        idx_refs = (i0, i1, i2, i3, i4, i5, i6)
        xv = (x0, x1, x2, x3, x4, x5, x6)
        rv = (r0, r1, r2, r3, r4, r5, r6)
        wid = lax.axis_index("s") * NC + lax.axis_index("c")
        base = wid * b_per_w

        pltpu.sync_copy(w_hbm, w_v)
        for t in range(NT):
            pltpu.sync_copy(idx_refs[t].at[pl.ds(base, b_per_w)], xv[t])
        descs = [
            pltpu.async_copy(tab_refs[t].at[xv[t]], rv[t], sem)
            for t in range(NT)
        ]
        for d in descs:
            d.wait()

        def body(r, carry):
            for c in (0, L):
                acc = w_v[0] * rv[0][r, pl.ds(c, L)]
                for t in range(1, NT):
                    acc = acc + w_v[t] * rv[t][r, pl.ds(c, L)]
                rv[0][r, pl.ds(c, L)] = acc
            return carry

        lax.fori_loop(0, b_per_w, body, 0)
        pltpu.sync_copy(rv[0], out_hbm.at[pl.ds(base, b_per_w)])

    return k(*tabs, *idxs, w_bcast)


def _mlp_body(x_ref, w1_ref, b1_ref, w2_ref, b2_ref, g_ref, be_ref, o_ref):
    x = x_ref[...]
    h = jnp.dot(x, w1_ref[...], preferred_element_type=jnp.float32,
                precision=lax.Precision.HIGHEST)
    h = jnp.maximum(h + b1_ref[...], 0.0)
    o = jnp.dot(h, w2_ref[...], preferred_element_type=jnp.float32,
                precision=lax.Precision.HIGHEST)
    o = o + b2_ref[...]
    mu = jnp.mean(o, axis=-1, keepdims=True)
    var = jnp.mean((o - mu) ** 2, axis=-1, keepdims=True)
    o_ref[...] = (o - mu) * lax.rsqrt(var + 1e-5) * g_ref[...] + be_ref[...]


@functools.partial(jax.jit, static_argnums=(0, 1))
def _mlp(B, blk, x, W1, b1, W2, b2, gamma, beta):
    H = W1.shape[1]
    ED = W2.shape[1]
    return pl.pallas_call(
        _mlp_body,
        grid=(B // blk,),
        in_specs=[
            pl.BlockSpec((blk, RD), lambda i: (i, 0)),
            pl.BlockSpec((RD, H), lambda i: (0, 0)),
            pl.BlockSpec((1, H), lambda i: (0, 0)),
            pl.BlockSpec((H, ED), lambda i: (0, 0)),
            pl.BlockSpec((1, ED), lambda i: (0, 0)),
            pl.BlockSpec((1, ED), lambda i: (0, 0)),
            pl.BlockSpec((1, ED), lambda i: (0, 0)),
        ],
        out_specs=pl.BlockSpec((blk, ED), lambda i: (i, 0)),
        out_shape=jax.ShapeDtypeStruct((B, ED), jnp.float32),
    )(x, W1, b1, W2, b2, gamma, beta)


def kernel(idx_phylum, idx_class, idx_order, idx_family, idx_genus,
           idx_species, tab_phylum, tab_class, tab_order, tab_family,
           tab_genus, tab_species, trophic_idx, trophic_tab, rank_weights,
           W1, b1, W2, b2, gamma, beta):
    B = idx_phylum.shape[0]
    idxs = [idx_phylum, idx_class, idx_order, idx_family, idx_genus,
            idx_species, trophic_idx]
    idxs = [i.astype(jnp.int32) for i in idxs]
    tabs = [tab_phylum, tab_class, tab_order, tab_family, tab_genus,
            tab_species, trophic_tab]
    w = jax.nn.softmax(rank_weights)
    wpad = jnp.zeros((8,), jnp.float32).at[:6].set(w).at[6].set(1.0)
    w_bcast = jnp.broadcast_to(wpad[:, None], (8, L))

    combined = _combine(B, tuple(tabs), tuple(idxs), w_bcast)
    out = _mlp(B, 2048, combined, W1, b1.reshape(1, -1), W2,
               b2.reshape(1, -1), gamma.reshape(1, -1), beta.reshape(1, -1))
    return out


# speedup vs baseline: 1.0032x; 1.0032x over previous
import functools

import jax
import jax.numpy as jnp
from jax import lax
from jax.experimental import pallas as pl
from jax.experimental.pallas import tpu as pltpu
import jax.experimental.pallas.tpu_sc as plsc

NC = 2
NS = 16
L = 16
NW = NC * NS

RD = 32
NT = 7


@functools.partial(jax.jit, static_argnums=(0,))
def _combine(B, tabs, idxs, w_bcast):
    b_per_w = B // NW
    mesh = plsc.VectorSubcoreMesh(core_axis_name="c", subcore_axis_name="s")

    @functools.partial(
        pl.kernel,
        out_type=jax.ShapeDtypeStruct((B, RD), jnp.float32),
        mesh=mesh,
        scratch_types=(
            [pltpu.VMEM((b_per_w,), jnp.int32) for _ in range(NT)]
            + [pltpu.VMEM((b_per_w, RD), jnp.float32) for _ in range(NT)]
            + [pltpu.VMEM((8, L), jnp.float32),
               pltpu.SemaphoreType.DMA, pltpu.SemaphoreType.DMA]
        ),
        compiler_params=pltpu.CompilerParams(use_tc_tiling_on_sc=False),
    )
    def k(t0, t1, t2, t3, t4, t5, t6, i0, i1, i2, i3, i4, i5, i6, w_hbm,
          out_hbm, x0, x1, x2, x3, x4, x5, x6, r0, r1, r2, r3, r4, r5, r6,
          w_v, sem, sem2):
        tab_refs = (t0, t1, t2, t3, t4, t5, t6)
        idx_refs = (i0, i1, i2, i3, i4, i5, i6)
        xv = (x0, x1, x2, x3, x4, x5, x6)
        rv = (r0, r1, r2, r3, r4, r5, r6)
        wid = lax.axis_index("s") * NC + lax.axis_index("c")
        base = wid * b_per_w

        idx_descs = [
            pltpu.async_copy(idx_refs[t].at[pl.ds(base, b_per_w)], xv[t], sem2)
            for t in range(NT)
        ]
        pltpu.sync_copy(w_hbm, w_v)
        for d in idx_descs:
            d.wait()
        descs = [
            pltpu.async_copy(tab_refs[t].at[xv[t]], rv[t], sem)
            for t in range(NT)
        ]
        for d in descs:
            d.wait()

        def body(g, carry):
            r0_ = g * 8
            for dr in range(8):
                r = r0_ + dr
                for c in (0, L):
                    acc = w_v[0] * rv[0][r, pl.ds(c, L)]
                    for t in range(1, NT):
                        acc = acc + w_v[t] * rv[t][r, pl.ds(c, L)]
                    rv[0][r, pl.ds(c, L)] = acc
            return carry

        lax.fori_loop(0, b_per_w // 8, body, 0)
        pltpu.sync_copy(rv[0], out_hbm.at[pl.ds(base, b_per_w)])

    return k(*tabs, *idxs, w_bcast)


def _mlp_body(x_ref, w1_ref, b1_ref, w2_ref, b2_ref, g_ref, be_ref, o_ref):
    x = x_ref[...]
    h = jnp.dot(x, w1_ref[...], preferred_element_type=jnp.float32,
                precision=lax.Precision.HIGHEST)
    h = jnp.maximum(h + b1_ref[...], 0.0)
    o = jnp.dot(h, w2_ref[...], preferred_element_type=jnp.float32,
                precision=lax.Precision.HIGHEST)
    o = o + b2_ref[...]
    mu = jnp.mean(o, axis=-1, keepdims=True)
    var = jnp.mean((o - mu) ** 2, axis=-1, keepdims=True)
    o_ref[...] = (o - mu) * lax.rsqrt(var + 1e-5) * g_ref[...] + be_ref[...]


@functools.partial(jax.jit, static_argnums=(0, 1))
def _mlp(B, blk, x, W1, b1, W2, b2, gamma, beta):
    H = W1.shape[1]
    ED = W2.shape[1]
    return pl.pallas_call(
        _mlp_body,
        grid=(B // blk,),
        in_specs=[
            pl.BlockSpec((blk, RD), lambda i: (i, 0)),
            pl.BlockSpec((RD, H), lambda i: (0, 0)),
            pl.BlockSpec((1, H), lambda i: (0, 0)),
            pl.BlockSpec((H, ED), lambda i: (0, 0)),
            pl.BlockSpec((1, ED), lambda i: (0, 0)),
            pl.BlockSpec((1, ED), lambda i: (0, 0)),
            pl.BlockSpec((1, ED), lambda i: (0, 0)),
        ],
        out_specs=pl.BlockSpec((blk, ED), lambda i: (i, 0)),
        out_shape=jax.ShapeDtypeStruct((B, ED), jnp.float32),
    )(x, W1, b1, W2, b2, gamma, beta)


def kernel(idx_phylum, idx_class, idx_order, idx_family, idx_genus,
           idx_species, tab_phylum, tab_class, tab_order, tab_family,
           tab_genus, tab_species, trophic_idx, trophic_tab, rank_weights,
           W1, b1, W2, b2, gamma, beta):
    B = idx_phylum.shape[0]
    idxs = [idx_phylum, idx_class, idx_order, idx_family, idx_genus,
            idx_species, trophic_idx]
    idxs = [i.astype(jnp.int32) for i in idxs]
    tabs = [tab_phylum, tab_class, tab_order, tab_family, tab_genus,
            tab_species, trophic_tab]
    w = jax.nn.softmax(rank_weights)
    wpad = jnp.zeros((8,), jnp.float32).at[:6].set(w).at[6].set(1.0)
    w_bcast = jnp.broadcast_to(wpad[:, None], (8, L))

    combined = _combine(B, tuple(tabs), tuple(idxs), w_bcast)
    out = _mlp(B, 2048, combined, W1, b1.reshape(1, -1), W2,
               b2.reshape(1, -1), gamma.reshape(1, -1), beta.reshape(1, -1))
    return out
